# Initial kernel scaffold; baseline (speedup 1.0000x reference)
import jax
import jax.numpy as jnp
from jax.experimental import pallas as pl


def kernel(x, weight):
    raise NotImplementedError("write your pallas kernel here")


# speedup vs baseline: 1.7610x; 1.7610x over previous
import functools

import jax
import jax.numpy as jnp
from jax import lax
from jax.experimental import pallas as pl
from jax.experimental.pallas import tpu as pltpu
from jax.experimental.pallas import tpu_sc as plsc

NUM_CORES = 2
NUM_SUBCORES = 16
NW = NUM_CORES * NUM_SUBCORES

D = 128
CHUNK = 128
EPS = 1e-4


def _rsqrt_vec(m):
    i = lax.bitcast_convert_type(m, jnp.int32)
    y = lax.bitcast_convert_type(jnp.int32(0x5F3759DF) - (i >> 1),
                                 jnp.float32)
    for _ in range(3):
        y = y * (1.5 - 0.5 * m * y * y)
    return y


def _sc_embed(n_total):
    per_w = n_total // NW
    n_chunks = per_w // CHUNK
    mesh = plsc.VectorSubcoreMesh(
        core_axis_name="c", subcore_axis_name="s",
        num_cores=NUM_CORES, num_subcores=NUM_SUBCORES)

    @functools.partial(
        pl.kernel,
        out_type=jax.ShapeDtypeStruct((n_total, D), jnp.float32),
        mesh=mesh,
        scratch_types=[
            pltpu.VMEM((n_chunks, CHUNK), jnp.int32),
            pltpu.VMEM((CHUNK, D), jnp.float32),
            pltpu.SemaphoreType.DMA,
        ],
    )
    def k(idx_hbm, table_hbm, out_hbm, idx_v, rows_v, sem):
        wid = lax.axis_index("s") * NUM_CORES + lax.axis_index("c")
        pltpu.sync_copy(idx_hbm.at[wid], idx_v)
        lanes = lax.iota(jnp.int32, 16)

        def chunk_body(j, carry):
            pltpu.async_copy(table_hbm.at[idx_v.at[j]], rows_v, sem).wait()

            def row_body(r, c):
                vs = [rows_v[r, pl.ds(16 * q, 16)] for q in range(8)]
                acc = vs[0] * vs[0]
                for q in range(1, 8):
                    acc = acc + vs[q] * vs[q]
                for sh in (8, 4, 2, 1):
                    acc = acc + acc.at[lanes ^ sh].get(
                        mode="promise_in_bounds")
                y = _rsqrt_vec(acc * (1.0 / D) + EPS)
                for q in range(8):
                    rows_v[r, pl.ds(16 * q, 16)] = vs[q] * y
                return c

            lax.fori_loop(0, CHUNK, row_body, 0)
            pltpu.sync_copy(
                rows_v, out_hbm.at[pl.ds(wid * per_w + j * CHUNK, CHUNK)])
            return carry

        lax.fori_loop(0, n_chunks, chunk_body, 0)

    return k


def kernel(x, weight):
    b, s = x.shape
    n_total = b * s
    idx = x.astype(jnp.int32).reshape(NW, (n_total // NW) // CHUNK, CHUNK)
    out = _sc_embed(n_total)(idx, weight)
    return out.reshape(b, s, weight.shape[1])
